# Initial kernel scaffold; baseline (speedup 1.0000x reference)
import jax
import jax.numpy as jnp
from jax.experimental import pallas as pl


def kernel(nodes, neigh_idx, self_table, neigh_table, W):
    raise NotImplementedError("write your pallas kernel here")


# speedup vs baseline: 2.1926x; 2.1926x over previous
import functools

import jax
import jax.numpy as jnp
import numpy as np
from jax import lax
from jax.experimental import pallas as pl
from jax.experimental.pallas import tpu as pltpu
from jax.experimental.pallas import tpu_sc as plsc

B = 50000
S = 10
F = 128
NC = 2
NS = 16
NW = NC * NS

CHUNK = 64
IDX_PER_CHUNK = CHUNK * S
SLABS = IDX_PER_CHUNK // 128
BP = 51200
RPT = BP // NW
NCHUNK = RPT // CHUNK
NIDX_ROWS_PER_TILE = RPT * S // 128
NIDX_ROWS_PER_CHUNK = IDX_PER_CHUNK // 128

_mesh = plsc.VectorSubcoreMesh(core_axis_name="c", subcore_axis_name="s")


@functools.partial(
    pl.kernel,
    mesh=_mesh,
    out_type=(
        jax.ShapeDtypeStruct((BP, F), jnp.float32),
        jax.ShapeDtypeStruct((BP, F), jnp.float32),
    ),
    scratch_types=[
        pltpu.VMEM((IDX_PER_CHUNK,), jnp.int32),
        pltpu.VMEM((CHUNK,), jnp.int32),
        pltpu.VMEM((SLABS, 128), jnp.int32),
        pltpu.VMEM((SLABS, 128), jnp.int32),
        pltpu.VMEM((IDX_PER_CHUNK, F), jnp.float32),
        pltpu.VMEM_SHARED((NS * CHUNK, F), jnp.float32),
        pltpu.VMEM((CHUNK, F), jnp.float32),
        pltpu.VMEM((CHUNK, F), jnp.float32),
        pltpu.SemaphoreType.DMA,
        pltpu.SemaphoreType.DMA,
    ],
)
def _sc_gather(self_tab, neigh_tab, sidx_hbm, nidx_hbm, rowmap_hbm, zeros_hbm,
               self_out, nsum_out,
               nidx_v, sidx_v, rowmap_v, rowmap_abs, gbuf, acc_sh, sbuf, zbuf,
               sem, sem2):
    sid = lax.axis_index("s")
    wid = sid * NC + lax.axis_index("c")
    pltpu.sync_copy(rowmap_hbm, rowmap_v)
    pltpu.sync_copy(zeros_hbm, zbuf)
    aoff = sid * CHUNK
    for j in range(SLABS):
        for c in range(8):
            sl = (pl.ds(j, 1), pl.ds(c * 16, 16))
            rowmap_abs.at[*sl][...] = rowmap_v.at[*sl][...] + aoff

    @pl.loop(0, NCHUNK)
    def _(k):
        base = wid * RPT + k * CHUNK
        ioff = base * S
        pltpu.sync_copy(nidx_hbm.at[pl.ds(ioff, IDX_PER_CHUNK)], nidx_v)
        pltpu.sync_copy(sidx_hbm.at[pl.ds(base, CHUNK)], sidx_v)
        gcp = [
            pltpu.async_copy(
                neigh_tab.at[nidx_v.at[pl.ds(j * 128, 128)]],
                gbuf.at[pl.ds(j * 128, 128)],
                sem,
            )
            for j in range(SLABS)
        ]
        scp = pltpu.async_copy(self_tab.at[sidx_v], sbuf, sem2)
        pltpu.sync_copy(zbuf, acc_sh.at[pl.ds(sid * CHUNK, CHUNK)])
        for cp in gcp:
            cp.wait()
        for j in range(SLABS):
            pltpu.sync_copy(
                gbuf.at[pl.ds(j * 128, 128)],
                acc_sh.at[rowmap_abs.at[j]],
                add=True,
            )
        pltpu.sync_copy(
            acc_sh.at[pl.ds(sid * CHUNK, CHUNK)],
            nsum_out.at[pl.ds(base, CHUNK)],
        )
        scp.wait()
        pltpu.sync_copy(sbuf, self_out.at[pl.ds(base, CHUNK)])


def _tc_matmul(s_ref, n_ref, w_ref, o_ref):
    x = jnp.concatenate([s_ref[...], n_ref[...]], axis=1)
    y = jnp.dot(x, w_ref[...], preferred_element_type=jnp.float32)
    o_ref[...] = jnp.maximum(y, 0.0)


_TC_BLK = 512


def kernel(nodes, neigh_idx, self_table, neigh_table, W):
    nodes = nodes.astype(jnp.int32)
    neigh_idx = neigh_idx.astype(jnp.int32)
    sidx = jnp.concatenate([nodes, jnp.zeros((BP - B,), jnp.int32)])
    nidx = jnp.concatenate(
        [neigh_idx.reshape(-1), jnp.zeros(((BP - B) * S,), jnp.int32)]
    )
    rowmap = jnp.asarray(
        (np.arange(IDX_PER_CHUNK, dtype=np.int32) // S).reshape(SLABS, 128)
    )
    zeros = jnp.zeros((CHUNK, F), jnp.float32)

    self_f, nsum = _sc_gather(self_table, neigh_table, sidx, nidx, rowmap, zeros)

    wt = jnp.concatenate([W[:, :F].T, W[:, F:].T * (1.0 / S)], axis=0)
    out = pl.pallas_call(
        _tc_matmul,
        grid=(BP // _TC_BLK,),
        in_specs=[
            pl.BlockSpec((_TC_BLK, F), lambda i: (i, 0)),
            pl.BlockSpec((_TC_BLK, F), lambda i: (i, 0)),
            pl.BlockSpec((2 * F, F), lambda i: (0, 0)),
        ],
        out_specs=pl.BlockSpec((_TC_BLK, F), lambda i: (i, 0)),
        out_shape=jax.ShapeDtypeStruct((BP, F), jnp.float32),
    )(self_f, nsum, wt)
    return out[:B]


# speedup vs baseline: 2.5910x; 1.1817x over previous
import functools

import jax
import jax.numpy as jnp
import numpy as np
from jax import lax
from jax.experimental import pallas as pl
from jax.experimental.pallas import tpu as pltpu
from jax.experimental.pallas import tpu_sc as plsc

B = 50000
S = 10
F = 128
NC = 2
NS = 16
NW = NC * NS

CHUNK = 64
IDX_PER_CHUNK = CHUNK * S
SLABS = IDX_PER_CHUNK // 128
BP = 51200
RPT = BP // NW
NCHUNK = RPT // CHUNK

_mesh = plsc.VectorSubcoreMesh(core_axis_name="c", subcore_axis_name="s")


@functools.partial(
    pl.kernel,
    mesh=_mesh,
    out_type=(
        jax.ShapeDtypeStruct((BP, F), jnp.float32),
        jax.ShapeDtypeStruct((BP, F), jnp.float32),
    ),
    scratch_types=[
        pltpu.VMEM((IDX_PER_CHUNK,), jnp.int32),
        pltpu.VMEM((IDX_PER_CHUNK,), jnp.int32),
        pltpu.VMEM((CHUNK,), jnp.int32),
        pltpu.VMEM((CHUNK,), jnp.int32),
        pltpu.VMEM((SLABS, 128), jnp.int32),
        pltpu.VMEM((SLABS, 128), jnp.int32),
        pltpu.VMEM((SLABS, 128), jnp.int32),
        pltpu.VMEM((IDX_PER_CHUNK, F), jnp.float32),
        pltpu.VMEM_SHARED((NS * 2 * CHUNK, F), jnp.float32),
        pltpu.VMEM((CHUNK, F), jnp.float32),
        pltpu.VMEM((CHUNK, F), jnp.float32),
        pltpu.VMEM((CHUNK, F), jnp.float32),
        pltpu.SemaphoreType.DMA,
        pltpu.SemaphoreType.DMA,
        pltpu.SemaphoreType.DMA,
        pltpu.SemaphoreType.DMA,
        pltpu.SemaphoreType.DMA,
        pltpu.SemaphoreType.DMA,
        pltpu.SemaphoreType.DMA,
        pltpu.SemaphoreType.DMA,
        pltpu.SemaphoreType.DMA,
    ],
)
def _sc_gather(self_tab, neigh_tab, sidx_hbm, nidx_hbm, rowmap_hbm, zeros_hbm,
               self_out, nsum_out,
               nidx0, nidx1, sidx0, sidx1, rowmap_v, rowabs0, rowabs1,
               gbuf, acc_sh, sbuf0, sbuf1, zbuf,
               gsem0, gsem1, gsem2, gsem3, gsem4,
               ssem, wsem, swsem, isem):
    nidx = (nidx0, nidx1)
    sidx = (sidx0, sidx1)
    sbuf = (sbuf0, sbuf1)
    rowabs = (rowabs0, rowabs1)
    gsem = (gsem0, gsem1, gsem2, gsem3, gsem4)

    sid = lax.axis_index("s")
    wid = sid * NC + lax.axis_index("c")
    pltpu.sync_copy(rowmap_hbm, rowmap_v)
    pltpu.sync_copy(zeros_hbm, zbuf)

    def acc_region(b):
        return acc_sh.at[pl.ds((sid * 2 + b) * CHUNK, CHUNK)]

    for b in range(2):
        aoff = (sid * 2 + b) * CHUNK
        for j in range(SLABS):
            for c in range(8):
                sl = (pl.ds(j, 1), pl.ds(c * 16, 16))
                rowabs[b].at[*sl][...] = rowmap_v.at[*sl][...] + aoff

    def idx_base(cur):
        return wid * RPT + cur * CHUNK

    def slab(j):
        return gbuf.at[pl.ds(j * 128, 128)]

    def issue_idx_prefetch(cur, b):
        pltpu.async_copy(
            nidx_hbm.at[pl.ds(idx_base(cur) * S, IDX_PER_CHUNK)], nidx[b], isem)
        pltpu.async_copy(
            sidx_hbm.at[pl.ds(idx_base(cur), CHUNK)], sidx[b], isem)

    def drain_idx_prefetch(b):
        pltpu.make_async_copy(
            nidx_hbm.at[pl.ds(0, IDX_PER_CHUNK)], nidx[b], isem).wait()
        pltpu.make_async_copy(
            sidx_hbm.at[pl.ds(0, CHUNK)], sidx[b], isem).wait()

    def issue_gathers(b):
        for j in range(SLABS):
            pltpu.async_copy(
                neigh_tab.at[nidx[b].at[pl.ds(j * 128, 128)]], slab(j), gsem[j])

    def issue_self_gather(b):
        pltpu.async_copy(self_tab.at[sidx[b]], sbuf[b], ssem)

    pltpu.sync_copy(nidx_hbm.at[pl.ds(idx_base(0) * S, IDX_PER_CHUNK)], nidx[0])
    pltpu.sync_copy(sidx_hbm.at[pl.ds(idx_base(0), CHUNK)], sidx[0])
    issue_gathers(0)
    issue_self_gather(0)
    pltpu.sync_copy(zbuf, acc_region(0))
    pltpu.sync_copy(zbuf, acc_region(1))

    @pl.loop(0, NCHUNK + 1, step=2)
    def _(k):
        for b in range(2):
            cur = k + b

            @pl.when(cur < NCHUNK)
            def _():
                base = idx_base(cur)
                have_next = cur + 1 < NCHUNK

                @pl.when(have_next)
                def _():
                    issue_idx_prefetch(cur + 1, b ^ 1)

                pltpu.make_async_copy(zeros_hbm, sbuf[b], ssem).wait()

                @pl.when(cur >= 1)
                def _():
                    pltpu.make_async_copy(zeros_hbm, sbuf[b ^ 1], swsem).wait()

                pltpu.async_copy(
                    sbuf[b], self_out.at[pl.ds(base, CHUNK)], swsem)

                @pl.when(cur >= 2)
                def _():
                    pltpu.make_async_copy(zeros_hbm, acc_region(b), wsem).wait()

                pltpu.sync_copy(zbuf, acc_region(b))

                @pl.when(have_next)
                def _():
                    drain_idx_prefetch(b ^ 1)
                    issue_self_gather(b ^ 1)

                for j in range(SLABS):
                    pltpu.make_async_copy(
                        neigh_tab.at[pl.ds(0, 128)], slab(j), gsem[j]).wait()
                    pltpu.sync_copy(slab(j), acc_sh.at[rowabs[b].at[j]],
                                    add=True)

                    @pl.when(have_next)
                    def _():
                        pltpu.async_copy(
                            neigh_tab.at[nidx[b ^ 1].at[pl.ds(j * 128, 128)]],
                            slab(j), gsem[j])

                pltpu.async_copy(
                    acc_region(b), nsum_out.at[pl.ds(base, CHUNK)], wsem)

    pltpu.make_async_copy(zeros_hbm, acc_region(0), wsem).wait()
    pltpu.make_async_copy(zeros_hbm, acc_region(1), wsem).wait()
    pltpu.make_async_copy(zeros_hbm, sbuf[0], swsem).wait()


def _tc_matmul(s_ref, n_ref, w_ref, o_ref):
    x = jnp.concatenate([s_ref[...], n_ref[...]], axis=1)
    y = jnp.dot(x, w_ref[...], preferred_element_type=jnp.float32)
    o_ref[...] = jnp.maximum(y, 0.0)


_TC_BLK = 512


def kernel(nodes, neigh_idx, self_table, neigh_table, W):
    nodes = nodes.astype(jnp.int32)
    neigh_idx = neigh_idx.astype(jnp.int32)
    sidx = jnp.concatenate([nodes, jnp.zeros((BP - B,), jnp.int32)])
    nidx = jnp.concatenate(
        [neigh_idx.reshape(-1), jnp.zeros(((BP - B) * S,), jnp.int32)]
    )
    rowmap = jnp.asarray(
        (np.arange(IDX_PER_CHUNK, dtype=np.int32) // S).reshape(SLABS, 128)
    )
    zeros = jnp.zeros((CHUNK, F), jnp.float32)

    self_f, nsum = _sc_gather(self_table, neigh_table, sidx, nidx, rowmap, zeros)

    wt = jnp.concatenate([W[:, :F].T, W[:, F:].T * (1.0 / S)], axis=0)
    out = pl.pallas_call(
        _tc_matmul,
        grid=(BP // _TC_BLK,),
        in_specs=[
            pl.BlockSpec((_TC_BLK, F), lambda i: (i, 0)),
            pl.BlockSpec((_TC_BLK, F), lambda i: (i, 0)),
            pl.BlockSpec((2 * F, F), lambda i: (0, 0)),
        ],
        out_specs=pl.BlockSpec((_TC_BLK, F), lambda i: (i, 0)),
        out_shape=jax.ShapeDtypeStruct((BP, F), jnp.float32),
    )(self_f, nsum, wt)
    return out[:B]


# speedup vs baseline: 2.5931x; 1.0008x over previous
import functools

import jax
import jax.numpy as jnp
import numpy as np
from jax import lax
from jax.experimental import pallas as pl
from jax.experimental.pallas import tpu as pltpu
from jax.experimental.pallas import tpu_sc as plsc

B = 50000
S = 10
F = 128
NC = 2
NS = 16
NW = NC * NS

CHUNK = 64
IDX_PER_CHUNK = CHUNK * S
SLABS = IDX_PER_CHUNK // 128
BP = 51200
RPT = BP // NW
NCHUNK = RPT // CHUNK

_mesh = plsc.VectorSubcoreMesh(core_axis_name="c", subcore_axis_name="s")


@functools.partial(
    pl.kernel,
    mesh=_mesh,
    out_type=(
        jax.ShapeDtypeStruct((BP, F), jnp.float32),
        jax.ShapeDtypeStruct((BP, F), jnp.float32),
    ),
    scratch_types=[
        pltpu.VMEM((IDX_PER_CHUNK,), jnp.int32),
        pltpu.VMEM((IDX_PER_CHUNK,), jnp.int32),
        pltpu.VMEM((CHUNK,), jnp.int32),
        pltpu.VMEM((CHUNK,), jnp.int32),
        pltpu.VMEM((SLABS, 128), jnp.int32),
        pltpu.VMEM((SLABS, 128), jnp.int32),
        pltpu.VMEM((SLABS, 128), jnp.int32),
        pltpu.VMEM((IDX_PER_CHUNK, F), jnp.float32),
        pltpu.VMEM_SHARED((NS * 2 * CHUNK, F), jnp.float32),
        pltpu.VMEM((CHUNK, F), jnp.float32),
        pltpu.VMEM((CHUNK, F), jnp.float32),
        pltpu.VMEM((CHUNK, F), jnp.float32),
        pltpu.SemaphoreType.DMA,
        pltpu.SemaphoreType.DMA,
        pltpu.SemaphoreType.DMA,
        pltpu.SemaphoreType.DMA,
        pltpu.SemaphoreType.DMA,
        pltpu.SemaphoreType.DMA,
        pltpu.SemaphoreType.DMA,
        pltpu.SemaphoreType.DMA,
        pltpu.SemaphoreType.DMA,
        pltpu.SemaphoreType.DMA,
        pltpu.SemaphoreType.DMA,
        pltpu.SemaphoreType.DMA,
        pltpu.SemaphoreType.DMA,
        pltpu.SemaphoreType.DMA,
        pltpu.SemaphoreType.DMA,
        pltpu.SemaphoreType.DMA,
    ],
)
def _sc_gather(self_tab, neigh_tab, sidx_hbm, nidx_hbm, rowmap_hbm, zeros_hbm,
               self_out, nsum_out,
               nidx0, nidx1, sidx0, sidx1, rowmap_v, rowabs0, rowabs1,
               gbuf, acc_sh, sbuf0, sbuf1, zbuf,
               gsem0, gsem1, gsem2, gsem3, gsem4,
               asem0, asem1, asem2, asem3, asem4,
               ssem, wsem0, wsem1, swsem0, swsem1, isem):
    nidx = (nidx0, nidx1)
    sidx = (sidx0, sidx1)
    sbuf = (sbuf0, sbuf1)
    rowabs = (rowabs0, rowabs1)
    gsem = (gsem0, gsem1, gsem2, gsem3, gsem4)
    asem = (asem0, asem1, asem2, asem3, asem4)
    wsem = (wsem0, wsem1)
    swsem = (swsem0, swsem1)

    sid = lax.axis_index("s")
    wid = sid * NC + lax.axis_index("c")
    pltpu.sync_copy(rowmap_hbm, rowmap_v)
    pltpu.sync_copy(zeros_hbm, zbuf)

    def acc_region(b):
        return acc_sh.at[pl.ds((sid * 2 + b) * CHUNK, CHUNK)]

    for b in range(2):
        aoff = (sid * 2 + b) * CHUNK
        for j in range(SLABS):
            for c in range(8):
                sl = (pl.ds(j, 1), pl.ds(c * 16, 16))
                rowabs[b].at[*sl][...] = rowmap_v.at[*sl][...] + aoff

    def idx_base(cur):
        return wid * RPT + cur * CHUNK

    def slab(j):
        return gbuf.at[pl.ds(j * 128, 128)]

    def issue_idx_prefetch(cur, b):
        pltpu.async_copy(
            nidx_hbm.at[pl.ds(idx_base(cur) * S, IDX_PER_CHUNK)], nidx[b], isem)
        pltpu.async_copy(
            sidx_hbm.at[pl.ds(idx_base(cur), CHUNK)], sidx[b], isem)

    def drain_idx_prefetch(b):
        pltpu.make_async_copy(
            nidx_hbm.at[pl.ds(0, IDX_PER_CHUNK)], nidx[b], isem).wait()
        pltpu.make_async_copy(
            sidx_hbm.at[pl.ds(0, CHUNK)], sidx[b], isem).wait()

    def issue_gathers(b):
        for j in range(SLABS):
            pltpu.async_copy(
                neigh_tab.at[nidx[b].at[pl.ds(j * 128, 128)]], slab(j), gsem[j])

    def issue_self_gather(b):
        pltpu.async_copy(self_tab.at[sidx[b]], sbuf[b], ssem)

    pltpu.sync_copy(nidx_hbm.at[pl.ds(idx_base(0) * S, IDX_PER_CHUNK)], nidx[0])
    pltpu.sync_copy(sidx_hbm.at[pl.ds(idx_base(0), CHUNK)], sidx[0])
    issue_gathers(0)
    issue_self_gather(0)
    pltpu.sync_copy(zbuf, acc_region(0))
    pltpu.sync_copy(zbuf, acc_region(1))

    @pl.loop(0, NCHUNK + 1, step=2)
    def _(k):
        for b in range(2):
            cur = k + b

            @pl.when(cur < NCHUNK)
            def _():
                base = idx_base(cur)
                have_next = cur + 1 < NCHUNK

                @pl.when(have_next)
                def _():
                    issue_idx_prefetch(cur + 1, b ^ 1)

                pltpu.make_async_copy(zeros_hbm, sbuf[b], ssem).wait()

                @pl.when(cur >= 1)
                def _():
                    pltpu.make_async_copy(
                        zeros_hbm, sbuf[b ^ 1], swsem[b ^ 1]).wait()

                pltpu.async_copy(
                    sbuf[b], self_out.at[pl.ds(base, CHUNK)], swsem[b])

                @pl.when(cur >= 2)
                def _():
                    pltpu.make_async_copy(
                        zeros_hbm, acc_region(b), wsem[b]).wait()

                pltpu.sync_copy(zbuf, acc_region(b))

                @pl.when(have_next)
                def _():
                    drain_idx_prefetch(b ^ 1)
                    issue_self_gather(b ^ 1)

                for wave in (0, 1):
                    js = range(wave, SLABS, 2)
                    for j in js:
                        pltpu.make_async_copy(
                            neigh_tab.at[pl.ds(0, 128)], slab(j),
                            gsem[j]).wait()
                        pltpu.async_copy(slab(j), acc_sh.at[rowabs[b].at[j]],
                                         asem[j], add=True)
                    for j in js:
                        pltpu.make_async_copy(
                            neigh_tab.at[pl.ds(0, 128)], slab(j),
                            asem[j]).wait()

                        @pl.when(have_next)
                        def _():
                            pltpu.async_copy(
                                neigh_tab.at[
                                    nidx[b ^ 1].at[pl.ds(j * 128, 128)]],
                                slab(j), gsem[j])

                pltpu.async_copy(
                    acc_region(b), nsum_out.at[pl.ds(base, CHUNK)], wsem[b])

    pltpu.make_async_copy(zeros_hbm, acc_region(0), wsem[0]).wait()
    pltpu.make_async_copy(zeros_hbm, acc_region(1), wsem[1]).wait()
    pltpu.make_async_copy(zeros_hbm, sbuf[0], swsem[0]).wait()


def _tc_matmul(s_ref, n_ref, w_ref, o_ref):
    x = jnp.concatenate([s_ref[...], n_ref[...]], axis=1)
    y = jnp.dot(x, w_ref[...], preferred_element_type=jnp.float32)
    o_ref[...] = jnp.maximum(y, 0.0)


_TC_BLK = 512


def kernel(nodes, neigh_idx, self_table, neigh_table, W):
    nodes = nodes.astype(jnp.int32)
    neigh_idx = neigh_idx.astype(jnp.int32)
    sidx = jnp.concatenate([nodes, jnp.zeros((BP - B,), jnp.int32)])
    nidx = jnp.concatenate(
        [neigh_idx.reshape(-1), jnp.zeros(((BP - B) * S,), jnp.int32)]
    )
    rowmap = jnp.asarray(
        (np.arange(IDX_PER_CHUNK, dtype=np.int32) // S).reshape(SLABS, 128)
    )
    zeros = jnp.zeros((CHUNK, F), jnp.float32)

    self_f, nsum = _sc_gather(self_table, neigh_table, sidx, nidx, rowmap, zeros)

    wt = jnp.concatenate([W[:, :F].T, W[:, F:].T * (1.0 / S)], axis=0)
    out = pl.pallas_call(
        _tc_matmul,
        grid=(BP // _TC_BLK,),
        in_specs=[
            pl.BlockSpec((_TC_BLK, F), lambda i: (i, 0)),
            pl.BlockSpec((_TC_BLK, F), lambda i: (i, 0)),
            pl.BlockSpec((2 * F, F), lambda i: (0, 0)),
        ],
        out_specs=pl.BlockSpec((_TC_BLK, F), lambda i: (i, 0)),
        out_shape=jax.ShapeDtypeStruct((BP, F), jnp.float32),
    )(self_f, nsum, wt)
    return out[:B]
